# baseline (speedup 1.0000x reference)
import functools

import jax
import jax.numpy as jnp
import numpy as np
from jax import lax
from jax.experimental import pallas as pl
from jax.experimental.pallas import tpu as pltpu
from jax.experimental.pallas import tpu_sc as plsc

B = 4096
F = 26
V = 100000
D = 64
DENSE = 13
H1 = 1024
H2 = 512

_N = B * F
_NC = 2
_NS = 16
_NW = _NC * _NS
_PER_W = _N // _NW
_CHUNK = 416
_NCHUNK = _PER_W // _CHUNK


def _gather_rows(flat_tables, flat_idx):
    mesh = plsc.VectorSubcoreMesh(core_axis_name="c", subcore_axis_name="s")

    @functools.partial(
        pl.kernel,
        out_type=jax.ShapeDtypeStruct((_N, D), jnp.float32),
        mesh=mesh,
        scratch_types=[
            pltpu.VMEM((_PER_W,), jnp.int32),
            pltpu.SemaphoreType.DMA,
        ],
    )
    def gather_kernel(tab_hbm, idx_hbm, out_hbm, idx_v, sem):
        wid = lax.axis_index("s") * _NC + lax.axis_index("c")
        base = wid * _PER_W
        pltpu.sync_copy(idx_hbm.at[pl.ds(base, _PER_W)], idx_v)

        def chunk_body(c, carry0):
            def grp(g, carryg):
                off = c * _CHUNK + g * 16
                vv = idx_v[pl.ds(off, 16)]
                for j in range(16):
                    s = vv[j]
                    pltpu.async_copy(
                        tab_hbm.at[pl.ds(s, 1), :],
                        out_hbm.at[pl.ds(base + off + j, 1), :],
                        sem,
                    )
                return carryg

            lax.fori_loop(0, _CHUNK // 16, grp, 0)
            pltpu.make_async_copy(
                tab_hbm.at[pl.ds(0, _CHUNK), :],
                out_hbm.at[pl.ds(base + c * _CHUNK, _CHUNK), :],
                sem,
            ).wait()
            return carry0

        lax.fori_loop(0, _NCHUNK, chunk_body, 0)

    return gather_kernel(flat_tables, flat_idx)


_BB = 512


def _mlp_body(x_ref, dense_ref, w1s_ref, w1d_ref, b1_ref, w2_ref, b2_ref,
              w3_ref, b3_ref, out_ref):
    h = jnp.dot(dense_ref[...], w1d_ref[...],
                preferred_element_type=jnp.float32)
    for f in range(F):
        h = h + jnp.dot(x_ref[:, f, :].astype(jnp.bfloat16), w1s_ref[f],
                        preferred_element_type=jnp.float32)
    h = jnp.maximum(h + b1_ref[...], 0.0).astype(jnp.bfloat16)
    h2 = jnp.dot(h, w2_ref[...], preferred_element_type=jnp.float32)
    h2 = jnp.maximum(h2 + b2_ref[...], 0.0)
    logit = jnp.sum(h2 * w3_ref[...], axis=1, keepdims=True) + b3_ref[...]
    out_ref[...] = jax.nn.sigmoid(logit)


def _mlp(x, dense, W1s, W1d, b1, W2, b2, w3row, b3):
    return pl.pallas_call(
        _mlp_body,
        grid=(B // _BB,),
        in_specs=[
            pl.BlockSpec((_BB, F, D), lambda i: (i, 0, 0)),
            pl.BlockSpec((_BB, DENSE), lambda i: (i, 0)),
            pl.BlockSpec((F, D, H1), lambda i: (0, 0, 0)),
            pl.BlockSpec((DENSE, H1), lambda i: (0, 0)),
            pl.BlockSpec((1, H1), lambda i: (0, 0)),
            pl.BlockSpec((H1, H2), lambda i: (0, 0)),
            pl.BlockSpec((1, H2), lambda i: (0, 0)),
            pl.BlockSpec((1, H2), lambda i: (0, 0)),
            pl.BlockSpec((1, 1), lambda i: (0, 0)),
        ],
        out_specs=pl.BlockSpec((_BB, 1), lambda i: (i, 0)),
        out_shape=jax.ShapeDtypeStruct((B, 1), jnp.float32),
    )(x, dense, W1s, W1d, b1, W2, b2, w3row, b3)


def kernel(sparse_ids, dense_feats, tables, W1, b1, W2, b2, W3, b3):
    flat_tables = tables.reshape(F * V, D)
    offs = (jnp.arange(F, dtype=jnp.int32) * V)[None, :]
    flat_idx = (sparse_ids.astype(jnp.int32) + offs).reshape(_N)

    emb = _gather_rows(flat_tables, flat_idx).reshape(B, F, D)

    W1s = W1[:F * D].reshape(F, D, H1).astype(jnp.bfloat16)
    W1d = W1[F * D:]
    W2b = W2.astype(jnp.bfloat16)

    return _mlp(emb, dense_feats, W1s, W1d, b1.reshape(1, H1), W2b,
                b2.reshape(1, H2), W3.reshape(1, H2), b3.reshape(1, 1))
